# baseline (device time: 107311 ns/iter reference)
import jax
import jax.numpy as jnp
from jax import lax
from jax.experimental import pallas as pl
from jax.experimental.pallas import tpu as pltpu

N_DEV = 4
N_HOP = N_DEV - 1
M_PER = 1024
N_OUT = 2048
HALF = N_OUT // 2
SUB = HALF // 2


_CHUNK_OFFS = (3, 1, 2, 0)
_OFF2SLOT = {off: j for j, off in enumerate(_CHUNK_OFFS)}


def kernel(x, w_mat):
    x = x.astype(jnp.bfloat16)
    w_mat = w_mat.astype(jnp.bfloat16)
    k_dim = x.shape[1]

    def body(x_ref, w_ref, out_ref, xv, comm_cw, comm_ccw, y_ref, amax_buf,
             xsem, ss_cw, rs_cw, ss_ccw, rs_ccw, a_ss, a_rs):
        pos = lax.axis_index("i")
        left = lax.rem(pos + N_DEV - 1, N_DEV)
        right = lax.rem(pos + 1, N_DEV)

        xcopies = []
        for j, off in enumerate(_CHUNK_OFFS):
            c = lax.rem(pos + off, N_DEV)
            cp = pltpu.make_async_copy(
                x_ref.at[pl.ds(c * M_PER, M_PER)], xv.at[j], xsem.at[j]
            )
            cp.start()
            xcopies.append(cp)

        barrier = pltpu.get_barrier_semaphore()
        for nbr in (left, right):
            pl.semaphore_signal(
                barrier, inc=1, device_id=(nbr,),
                device_id_type=pl.DeviceIdType.MESH,
            )
        pl.semaphore_wait(barrier, 2)

        w_loc = w_ref[...]
        comm = {"cw": comm_cw, "ccw": comm_ccw}
        ss = {"cw": ss_cw, "ccw": ss_ccw}
        rs = {"cw": rs_cw, "ccw": rs_ccw}
        peer = {"cw": right, "ccw": left}
        col0 = {"cw": 0, "ccw": HALF}

        xwaited = set()

        def xc(off):
            j = _OFF2SLOT[off % N_DEV]
            if j not in xwaited:
                xcopies[j].wait()
                xwaited.add(j)
            return xv[j, :, :]

        def psub(off, d, s):
            lo = col0[d] + s * SUB
            return jnp.dot(
                xc(off), w_loc[:, lo:lo + SUB],
                preferred_element_type=jnp.float32,
            )

        def make_rdma(d, h, s):
            return pltpu.make_async_remote_copy(
                src_ref=comm[d].at[h, s],
                dst_ref=comm[d].at[h + 1, s],
                send_sem=ss[d].at[h, s],
                recv_sem=rs[d].at[h, s],
                device_id=(peer[d],),
                device_id_type=pl.DeviceIdType.MESH,
            )

        def off_arr(d, h):
            return (N_DEV - 2 - h) if d == "cw" else (2 + h)

        sends = []

        off_seed = {"cw": N_DEV - 1, "ccw": 1}
        for s in range(2):
            for d in ("cw", "ccw"):
                comm[d][0, s] = psub(off_seed[d], d, s).astype(jnp.bfloat16)
                r = make_rdma(d, 0, s)
                r.start()
                sends.append(r)

        acc_final = {}
        for h in range(N_HOP):
            p = {(d, s): psub(off_arr(d, h), d, s)
                 for s in range(2) for d in ("cw", "ccw")}
            for s in range(2):
                for d in ("cw", "ccw"):
                    make_rdma(d, h, s).wait_recv()
                    acc = comm[d][h + 1, s].astype(jnp.float32) + p[(d, s)]
                    if h < N_HOP - 1:
                        comm[d][h + 1, s] = acc.astype(jnp.bfloat16)
                        r = make_rdma(d, h + 1, s)
                        r.start()
                        sends.append(r)
                    else:
                        acc_final[(d, s)] = acc

        lamax = jnp.float32(0)
        for (d, s), acc in acc_final.items():
            lo = col0[d] + s * SUB
            y_ref[:, lo:lo + SUB] = acc
            lamax = jnp.maximum(lamax, jnp.max(jnp.abs(acc)))

        amax_buf[pl.ds(pos, 1), :] = jnp.full((1, 128), lamax, jnp.float32)
        for k in range(1, N_DEV):
            tgt = lax.rem(pos + k, N_DEV)
            r = pltpu.make_async_remote_copy(
                src_ref=amax_buf.at[pl.ds(pos, 1)],
                dst_ref=amax_buf.at[pl.ds(pos, 1)],
                send_sem=a_ss.at[k - 1],
                recv_sem=a_rs.at[k - 1],
                device_id=(tgt,),
                device_id_type=pl.DeviceIdType.MESH,
            )
            r.start()
            sends.append(r)
        for k in range(1, N_DEV):
            src_pos = lax.rem(pos + N_DEV - k, N_DEV)
            r = pltpu.make_async_remote_copy(
                src_ref=amax_buf.at[pl.ds(pos, 1)],
                dst_ref=amax_buf.at[pl.ds(src_pos, 1)],
                send_sem=a_ss.at[k - 1],
                recv_sem=a_rs.at[k - 1],
                device_id=(left,),
                device_id_type=pl.DeviceIdType.MESH,
            )
            r.wait_recv()
        for r in sends:
            r.wait_send()

        gmax = jnp.max(amax_buf[...])

        inv = 448.0 / gmax
        q = jnp.clip(y_ref[...] * inv, -448.0, 448.0).astype(jnp.float8_e4m3fn)
        out_ref[...] = q.astype(jnp.float32) * (gmax / 448.0)

    return pl.pallas_call(
        body,
        out_shape=jax.ShapeDtypeStruct((M_PER, N_OUT), jnp.float32),
        in_specs=[
            pl.BlockSpec(memory_space=pl.ANY),
            pl.BlockSpec(memory_space=pltpu.VMEM),
        ],
        out_specs=pl.BlockSpec(memory_space=pltpu.VMEM),
        scratch_shapes=[
            pltpu.VMEM((N_DEV, M_PER, k_dim), jnp.bfloat16),
            pltpu.VMEM((N_DEV, 2, M_PER, SUB), jnp.bfloat16),
            pltpu.VMEM((N_DEV, 2, M_PER, SUB), jnp.bfloat16),
            pltpu.VMEM((M_PER, N_OUT), jnp.float32),
            pltpu.VMEM((N_DEV, 128), jnp.float32),
            pltpu.SemaphoreType.DMA((N_DEV,)),
            pltpu.SemaphoreType.DMA((N_HOP, 2)),
            pltpu.SemaphoreType.DMA((N_HOP, 2)),
            pltpu.SemaphoreType.DMA((N_HOP, 2)),
            pltpu.SemaphoreType.DMA((N_HOP, 2)),
            pltpu.SemaphoreType.DMA((N_DEV - 1,)),
            pltpu.SemaphoreType.DMA((N_DEV - 1,)),
        ],
        compiler_params=pltpu.CompilerParams(
            collective_id=0,
            vmem_limit_bytes=100 * 1024 * 1024,
        ),
    )(x, w_mat)


# device time: 93688 ns/iter; 1.1454x vs baseline; 1.1454x over previous
import jax
import jax.numpy as jnp
from jax import lax
from jax.experimental import pallas as pl
from jax.experimental.pallas import tpu as pltpu

N_DEV = 4
N_HOP = N_DEV - 1
M_PER = 1024
N_OUT = 2048
HALF = N_OUT // 2
NSUB = 4
SUB = HALF // NSUB


def kernel(x, w_mat):
    x = x.astype(jnp.bfloat16)
    w_mat = w_mat.astype(jnp.bfloat16)

    def body(x_ref, w_ref, out_ref, comm_cw, comm_ccw, y_ref, amax_buf,
             ss_cw, rs_cw, ss_ccw, rs_ccw, a_ss, a_rs):
        pos = lax.axis_index("i")
        left = lax.rem(pos + N_DEV - 1, N_DEV)
        right = lax.rem(pos + 1, N_DEV)

        barrier = pltpu.get_barrier_semaphore()
        for nbr in (left, right):
            pl.semaphore_signal(
                barrier, inc=1, device_id=(nbr,),
                device_id_type=pl.DeviceIdType.MESH,
            )
        pl.semaphore_wait(barrier, 2)

        w_loc = w_ref[...]
        comm = {"cw": comm_cw, "ccw": comm_ccw}
        ss = {"cw": ss_cw, "ccw": ss_ccw}
        rs = {"cw": rs_cw, "ccw": rs_ccw}
        peer = {"cw": right, "ccw": left}
        col0 = {"cw": 0, "ccw": HALF}

        def xc(c):
            return x_ref[pl.ds(c * M_PER, M_PER), :]

        def psub(c, d, s):
            lo = col0[d] + s * SUB
            return jnp.dot(
                xc(c), w_loc[:, lo:lo + SUB],
                preferred_element_type=jnp.float32,
            )

        def make_rdma(d, h, s):
            return pltpu.make_async_remote_copy(
                src_ref=comm[d].at[h, s],
                dst_ref=comm[d].at[h + 1, s],
                send_sem=ss[d].at[h, s],
                recv_sem=rs[d].at[h, s],
                device_id=(peer[d],),
                device_id_type=pl.DeviceIdType.MESH,
            )

        def c_arr(d, h):
            off = (N_DEV - 2 - h) if d == "cw" else (2 + h)
            return lax.rem(pos + off, N_DEV)

        sends = []

        c_seed = {"cw": lax.rem(pos + N_DEV - 1, N_DEV),
                  "ccw": lax.rem(pos + 1, N_DEV)}
        for s in range(NSUB):
            for d in ("cw", "ccw"):
                comm[d][0, s] = psub(c_seed[d], d, s).astype(jnp.bfloat16)
                r = make_rdma(d, 0, s)
                r.start()
                sends.append(r)

        acc_final = {}
        for h in range(N_HOP):
            p = {(d, s): psub(c_arr(d, h), d, s)
                 for s in range(NSUB) for d in ("cw", "ccw")}
            for s in range(NSUB):
                for d in ("cw", "ccw"):
                    make_rdma(d, h, s).wait_recv()
                    acc = comm[d][h + 1, s].astype(jnp.float32) + p[(d, s)]
                    if h < N_HOP - 1:
                        comm[d][h + 1, s] = acc.astype(jnp.bfloat16)
                        r = make_rdma(d, h + 1, s)
                        r.start()
                        sends.append(r)
                    else:
                        acc_final[(d, s)] = acc

        lamax = jnp.float32(0)
        for (d, s), acc in acc_final.items():
            lo = col0[d] + s * SUB
            y_ref[:, lo:lo + SUB] = acc
            lamax = jnp.maximum(lamax, jnp.max(jnp.abs(acc)))

        amax_buf[pl.ds(pos, 1), :] = jnp.full((1, 128), lamax, jnp.float32)
        for k in range(1, N_DEV):
            tgt = lax.rem(pos + k, N_DEV)
            r = pltpu.make_async_remote_copy(
                src_ref=amax_buf.at[pl.ds(pos, 1)],
                dst_ref=amax_buf.at[pl.ds(pos, 1)],
                send_sem=a_ss.at[k - 1],
                recv_sem=a_rs.at[k - 1],
                device_id=(tgt,),
                device_id_type=pl.DeviceIdType.MESH,
            )
            r.start()
            sends.append(r)
        for k in range(1, N_DEV):
            src_pos = lax.rem(pos + N_DEV - k, N_DEV)
            r = pltpu.make_async_remote_copy(
                src_ref=amax_buf.at[pl.ds(pos, 1)],
                dst_ref=amax_buf.at[pl.ds(src_pos, 1)],
                send_sem=a_ss.at[k - 1],
                recv_sem=a_rs.at[k - 1],
                device_id=(left,),
                device_id_type=pl.DeviceIdType.MESH,
            )
            r.wait_recv()
        for r in sends:
            r.wait_send()

        gmax = jnp.max(amax_buf[...])

        inv = 448.0 / gmax
        q = jnp.clip(y_ref[...] * inv, -448.0, 448.0).astype(jnp.float8_e4m3fn)
        out_ref[...] = q.astype(jnp.float32) * (gmax / 448.0)

    return pl.pallas_call(
        body,
        out_shape=jax.ShapeDtypeStruct((M_PER, N_OUT), jnp.float32),
        in_specs=[
            pl.BlockSpec(memory_space=pltpu.VMEM),
            pl.BlockSpec(memory_space=pltpu.VMEM),
        ],
        out_specs=pl.BlockSpec(memory_space=pltpu.VMEM),
        scratch_shapes=[
            pltpu.VMEM((N_DEV, NSUB, M_PER, SUB), jnp.bfloat16),
            pltpu.VMEM((N_DEV, NSUB, M_PER, SUB), jnp.bfloat16),
            pltpu.VMEM((M_PER, N_OUT), jnp.float32),
            pltpu.VMEM((N_DEV, 128), jnp.float32),
            pltpu.SemaphoreType.DMA((N_HOP, NSUB)),
            pltpu.SemaphoreType.DMA((N_HOP, NSUB)),
            pltpu.SemaphoreType.DMA((N_HOP, NSUB)),
            pltpu.SemaphoreType.DMA((N_HOP, NSUB)),
            pltpu.SemaphoreType.DMA((N_DEV - 1,)),
            pltpu.SemaphoreType.DMA((N_DEV - 1,)),
        ],
        compiler_params=pltpu.CompilerParams(collective_id=0),
    )(x, w_mat)


# device time: 93568 ns/iter; 1.1469x vs baseline; 1.0013x over previous
import jax
import jax.numpy as jnp
from jax import lax
from jax.experimental import pallas as pl
from jax.experimental.pallas import tpu as pltpu

N_DEV = 4
N_HOP = N_DEV - 1
M_PER = 1024
N_OUT = 2048
HALF = N_OUT // 2
NSUB = 2
SUB = HALF // NSUB


def kernel(x, w_mat):
    x = x.astype(jnp.bfloat16)
    w_mat = w_mat.astype(jnp.bfloat16)

    def body(x_ref, w_ref, out_ref, comm_cw, comm_ccw, y_ref, amax_buf,
             ss_cw, rs_cw, ss_ccw, rs_ccw, a_ss, a_rs):
        pos = lax.axis_index("i")
        left = lax.rem(pos + N_DEV - 1, N_DEV)
        right = lax.rem(pos + 1, N_DEV)

        barrier = pltpu.get_barrier_semaphore()
        for nbr in (left, right):
            pl.semaphore_signal(
                barrier, inc=1, device_id=(nbr,),
                device_id_type=pl.DeviceIdType.MESH,
            )
        pl.semaphore_wait(barrier, 2)

        w_loc = w_ref[...]
        comm = {"cw": comm_cw, "ccw": comm_ccw}
        ss = {"cw": ss_cw, "ccw": ss_ccw}
        rs = {"cw": rs_cw, "ccw": rs_ccw}
        peer = {"cw": right, "ccw": left}
        col0 = {"cw": 0, "ccw": HALF}

        def xc(c):
            return x_ref[pl.ds(c * M_PER, M_PER), :]

        def psub(c, d, s):
            lo = col0[d] + s * SUB
            return jnp.dot(
                xc(c), w_loc[:, lo:lo + SUB],
                preferred_element_type=jnp.float32,
            )

        def make_rdma(d, h, s):
            return pltpu.make_async_remote_copy(
                src_ref=comm[d].at[h, s],
                dst_ref=comm[d].at[h + 1, s],
                send_sem=ss[d].at[h, s],
                recv_sem=rs[d].at[h, s],
                device_id=(peer[d],),
                device_id_type=pl.DeviceIdType.MESH,
            )

        def c_arr(d, h):
            off = (N_DEV - 2 - h) if d == "cw" else (2 + h)
            return lax.rem(pos + off, N_DEV)

        sends = []

        c_seed = {"cw": lax.rem(pos + N_DEV - 1, N_DEV),
                  "ccw": lax.rem(pos + 1, N_DEV)}
        for s in range(NSUB):
            for d in ("cw", "ccw"):
                comm[d][0, s] = psub(c_seed[d], d, s).astype(jnp.bfloat16)
                r = make_rdma(d, 0, s)
                r.start()
                sends.append(r)

        acc_final = {}
        for h in range(N_HOP):
            p = {(d, s): psub(c_arr(d, h), d, s)
                 for s in range(NSUB) for d in ("cw", "ccw")}
            for s in range(NSUB):
                for d in ("cw", "ccw"):
                    make_rdma(d, h, s).wait_recv()
                    acc = comm[d][h + 1, s].astype(jnp.float32) + p[(d, s)]
                    if h < N_HOP - 1:
                        comm[d][h + 1, s] = acc.astype(jnp.bfloat16)
                        r = make_rdma(d, h + 1, s)
                        r.start()
                        sends.append(r)
                    else:
                        acc_final[(d, s)] = acc

        lamax = jnp.float32(0)
        for (d, s), acc in acc_final.items():
            lo = col0[d] + s * SUB
            y_ref[:, lo:lo + SUB] = acc
            lamax = jnp.maximum(lamax, jnp.max(jnp.abs(acc)))

        amax_buf[pl.ds(pos, 1), :] = jnp.full((1, 128), lamax, jnp.float32)
        for k in range(1, N_DEV):
            tgt = lax.rem(pos + k, N_DEV)
            r = pltpu.make_async_remote_copy(
                src_ref=amax_buf.at[pl.ds(pos, 1)],
                dst_ref=amax_buf.at[pl.ds(pos, 1)],
                send_sem=a_ss.at[k - 1],
                recv_sem=a_rs.at[k - 1],
                device_id=(tgt,),
                device_id_type=pl.DeviceIdType.MESH,
            )
            r.start()
            sends.append(r)
        for k in range(1, N_DEV):
            src_pos = lax.rem(pos + N_DEV - k, N_DEV)
            r = pltpu.make_async_remote_copy(
                src_ref=amax_buf.at[pl.ds(pos, 1)],
                dst_ref=amax_buf.at[pl.ds(src_pos, 1)],
                send_sem=a_ss.at[k - 1],
                recv_sem=a_rs.at[k - 1],
                device_id=(left,),
                device_id_type=pl.DeviceIdType.MESH,
            )
            r.wait_recv()
        for r in sends:
            r.wait_send()

        gmax = jnp.max(amax_buf[...])

        inv = 448.0 / gmax
        q = jnp.clip(y_ref[...] * inv, -448.0, 448.0).astype(jnp.float8_e4m3fn)
        out_ref[...] = q.astype(jnp.float32) * (gmax / 448.0)

    return pl.pallas_call(
        body,
        out_shape=jax.ShapeDtypeStruct((M_PER, N_OUT), jnp.float32),
        in_specs=[
            pl.BlockSpec(memory_space=pltpu.VMEM),
            pl.BlockSpec(memory_space=pltpu.VMEM),
        ],
        out_specs=pl.BlockSpec(memory_space=pltpu.VMEM),
        scratch_shapes=[
            pltpu.VMEM((N_DEV, NSUB, M_PER, SUB), jnp.bfloat16),
            pltpu.VMEM((N_DEV, NSUB, M_PER, SUB), jnp.bfloat16),
            pltpu.VMEM((M_PER, N_OUT), jnp.float32),
            pltpu.VMEM((N_DEV, 128), jnp.float32),
            pltpu.SemaphoreType.DMA((N_HOP, NSUB)),
            pltpu.SemaphoreType.DMA((N_HOP, NSUB)),
            pltpu.SemaphoreType.DMA((N_HOP, NSUB)),
            pltpu.SemaphoreType.DMA((N_HOP, NSUB)),
            pltpu.SemaphoreType.DMA((N_DEV - 1,)),
            pltpu.SemaphoreType.DMA((N_DEV - 1,)),
        ],
        compiler_params=pltpu.CompilerParams(collective_id=0),
    )(x, w_mat)
